# baseline (speedup 1.0000x reference)
import functools

import jax
import jax.numpy as jnp
from jax.experimental import pallas as pl
from jax.experimental.pallas import tpu as pltpu
from jax.experimental.pallas import tpu_sc as plsc

_MESH = plsc.ScalarSubcoreMesh(axis_name="c", num_cores=1)


@functools.partial(
    pl.kernel,
    out_type=jax.ShapeDtypeStruct((4, 12), jnp.float32),
    mesh=_MESH,
    scratch_types=[pltpu.SemaphoreType.DMA],
)
def _scatter_sc(x_hbm, z_hbm, out_hbm, sem):
    pltpu.async_copy(z_hbm, out_hbm.at[0], sem)
    pltpu.async_copy(x_hbm, out_hbm.at[1], sem)
    pltpu.async_copy(x_hbm, out_hbm.at[2], sem)
    pltpu.async_copy(z_hbm, out_hbm.at[3], sem)
    pltpu.make_async_copy(out_hbm, out_hbm, sem).wait()


def kernel(x):
    zeros = jnp.zeros((12,), jnp.float32)
    return _scatter_sc(x.reshape(12), zeros).reshape(4, 2, 2, 3)
